# baseline (speedup 1.0000x reference)
import functools

import jax
import jax.numpy as jnp
from jax import lax
from jax.experimental import pallas as pl
from jax.experimental.pallas import tpu as pltpu
from jax.experimental.pallas import tpu_sc as plsc

N = 50000
H = 256
HALF = 128
S = 512

BR = 8192
NBLK = 7
NPAD = NBLK * BR

NW = 16
CH = NPAD // NW
NV = CH // 16
SEGP = 528
RPW = S // NW


NSUB = 2
SR = BR // NSUB


def _fused_body(x_ref, w1_ref, b1_ref, w2_ref, b2_ref, b_ref, e_ref, p_ref):
    i = pl.program_id(0)

    @pl.when(i == 0)
    def _():
        p_ref[...] = jnp.zeros_like(p_ref)

    acc = jnp.zeros((S, H), jnp.float32)
    for k in range(NSUB):
        xb = x_ref[pl.ds(k * SR, SR), :]
        h = jnp.tanh(jnp.dot(xb, w1_ref[...],
                             preferred_element_type=jnp.float32) + b1_ref[...])
        sc = jnp.sum(h * w2_ref[...], axis=1, keepdims=True) + b2_ref[...]
        e = jnp.exp(sc)
        e_ref[pl.ds(k * SR, SR), :] = e
        xe = (xb * e).astype(jnp.bfloat16)
        b = b_ref[0, :, pl.ds(k * SR, SR)]
        seg = jax.lax.broadcasted_iota(jnp.int32, (S, SR), 0)
        onehot = (b == seg).astype(jnp.bfloat16)
        acc = acc + jnp.dot(onehot, xe, preferred_element_type=jnp.float32)
    p_ref[...] += acc


def _tc_fused(x, W1, b1r, W2r, b2r, b3d):
    return pl.pallas_call(
        _fused_body,
        grid=(NBLK,),
        in_specs=[
            pl.BlockSpec((BR, H), lambda i: (i, 0)),
            pl.BlockSpec((H, HALF), lambda i: (0, 0)),
            pl.BlockSpec((1, HALF), lambda i: (0, 0)),
            pl.BlockSpec((1, HALF), lambda i: (0, 0)),
            pl.BlockSpec((1, 1), lambda i: (0, 0)),
            pl.BlockSpec((1, 1, BR), lambda i: (i, 0, 0)),
        ],
        out_specs=[
            pl.BlockSpec((BR, 1), lambda i: (i, 0)),
            pl.BlockSpec((S, H), lambda i: (0, 0)),
        ],
        out_shape=[
            jax.ShapeDtypeStruct((NPAD, 1), jnp.float32),
            jax.ShapeDtypeStruct((S, H), jnp.float32),
        ],
    )(x, W1, b1r, W2r, b2r, b3d)


def _sc_body(e_hbm, batch_hbm, p_hbm, w_hbm, xp_hbm,
             e_v, b_v, acc_v, st_v, gd_v, p_v, shd):
    c = lax.axis_index("c")
    s = lax.axis_index("s")

    @pl.when(c == 0)
    def _work():
        base = s * CH
        pltpu.sync_copy(e_hbm.at[pl.ds(base, CH)], e_v)
        pltpu.sync_copy(batch_hbm.at[pl.ds(base, CH)], b_v)
        pltpu.sync_copy(p_hbm.at[pl.ds(s * (RPW * H), RPW * H)], p_v)

        lane = lax.iota(jnp.int32, 16)
        zero = jnp.zeros((16,), jnp.float32)
        lane_base = lane * SEGP

        def init_d(j, carry):
            off = pl.multiple_of(j * 16, 16)
            for t in range(NW):
                acc_v[pl.ds(t * SEGP + off, 16)] = zero
            return carry
        lax.fori_loop(0, SEGP // 16, init_d, 0)

        def p_acc(j, carry):
            off = pl.multiple_of(j * 16, 16)
            ev = e_v[pl.ds(off, 16)]
            bv = b_v[pl.ds(off, 16)]
            idx = lane_base + bv
            cur = plsc.load_gather(acc_v, [idx])
            plsc.store_scatter(acc_v, [idx], cur + ev)
            return carry
        lax.fori_loop(0, NV, p_acc, 0)

        def red_loc(j, carry):
            off = pl.multiple_of(j * 16, 16)
            acc = acc_v[pl.ds(off, 16)]
            for t in range(1, NW):
                acc = acc + acc_v[pl.ds(t * SEGP + off, 16)]
            gd_v[pl.ds(off, 16)] = acc
            return carry
        lax.fori_loop(0, SEGP // 16, red_loc, 0)

        pltpu.sync_copy(gd_v, shd.at[s])
        plsc.subcore_barrier()
        pltpu.sync_copy(shd, st_v)

        def red_glob(j, carry):
            off = pl.multiple_of(j * 16, 16)
            acc = st_v[0, pl.ds(off, 16)]
            for t in range(1, NW):
                acc = acc + st_v[t, pl.ds(off, 16)]
            gd_v[pl.ds(off, 16)] = acc
            return carry
        lax.fori_loop(0, SEGP // 16, red_glob, 0)

        def p_w(j, carry):
            off = pl.multiple_of(j * 16, 16)
            ev = e_v[pl.ds(off, 16)]
            bv = b_v[pl.ds(off, 16)]
            gd = plsc.load_gather(gd_v, [bv])
            wv = jnp.where(bv >= S, 0.0, ev / gd)
            e_v[pl.ds(off, 16)] = wv
            return carry
        lax.fori_loop(0, NV, p_w, 0)
        pltpu.sync_copy(e_v, w_hbm.at[pl.ds(base, CH)])

        for k in range(RPW):
            r = jnp.zeros((16,), jnp.int32) + (s * RPW + k)
            dvec = plsc.load_gather(gd_v, [r])
            scale = jnp.where(dvec > 0.0, 1.0 / dvec, 0.0)
            for t in range(H // 16):
                off = k * H + t * 16
                p_v[pl.ds(off, 16)] = p_v[pl.ds(off, 16)] * scale
        pltpu.sync_copy(p_v, xp_hbm.at[pl.ds(s * (RPW * H), RPW * H)])


@functools.cache
def _get_sc_kernel():
    return functools.partial(
        pl.kernel,
        mesh=plsc.VectorSubcoreMesh(core_axis_name="c", subcore_axis_name="s"),
        out_type=(jax.ShapeDtypeStruct((NPAD,), jnp.float32),
                  jax.ShapeDtypeStruct((S * H,), jnp.float32)),
        compiler_params=pltpu.CompilerParams(needs_layout_passes=False,
                                             use_tc_tiling_on_sc=False),
        scratch_types=[
            pltpu.VMEM((CH,), jnp.float32),
            pltpu.VMEM((CH,), jnp.int32),
            pltpu.VMEM((NW * SEGP,), jnp.float32),
            pltpu.VMEM((NW, SEGP), jnp.float32),
            pltpu.VMEM((SEGP,), jnp.float32),
            pltpu.VMEM((RPW * H,), jnp.float32),
            pltpu.VMEM_SHARED((NW, SEGP), jnp.float32),
        ],
    )(_sc_body)


def kernel(x, batch, W1, b1, W2, b2):
    b1r = b1.reshape(1, HALF)
    W2r = W2.reshape(1, HALF)
    b2r = b2.reshape(1, 1)
    batch_p = jnp.concatenate(
        [batch.astype(jnp.int32), jnp.full((NPAD - N,), S, jnp.int32)])
    e, P = _tc_fused(x, W1, b1r, W2r, b2r, batch_p.reshape(NBLK, 1, BR))
    weights, xp = _get_sc_kernel()(e.reshape(NPAD), batch_p,
                                   P.reshape(S * H))
    return xp.reshape(S, H), weights[:N]


# speedup vs baseline: 1.0169x; 1.0169x over previous
import functools

import jax
import jax.numpy as jnp
from jax import lax
from jax.experimental import pallas as pl
from jax.experimental.pallas import tpu as pltpu
from jax.experimental.pallas import tpu_sc as plsc

N = 50000
H = 256
HALF = 128
S = 512

BR = 4096
NBLK = 13
NPAD = NBLK * BR

NW = 16
CH = NPAD // NW
NV = CH // 16
SEGP = 528
RPW = S // NW


NSUB = 2
SR = BR // NSUB


def _fused_body(x_ref, w1_ref, b1_ref, w2_ref, b2_ref, b_ref, e_ref, p_ref):
    i = pl.program_id(0)

    @pl.when(i == 0)
    def _():
        p_ref[...] = jnp.zeros_like(p_ref)

    acc = jnp.zeros((S, H), jnp.float32)
    for k in range(NSUB):
        xb = x_ref[pl.ds(k * SR, SR), :]
        h = jnp.tanh(jnp.dot(xb, w1_ref[...],
                             preferred_element_type=jnp.float32) + b1_ref[...])
        sc = jnp.sum(h * w2_ref[...], axis=1, keepdims=True) + b2_ref[...]
        e = jnp.exp(sc)
        e_ref[pl.ds(k * SR, SR), :] = e
        xe = (xb * e).astype(jnp.bfloat16)
        b = b_ref[0, :, pl.ds(k * SR, SR)]
        seg = jax.lax.broadcasted_iota(jnp.int32, (S, SR), 0)
        onehot = (b == seg).astype(jnp.bfloat16)
        acc = acc + jnp.dot(onehot, xe, preferred_element_type=jnp.float32)
    p_ref[...] += acc


def _tc_fused(x, W1, b1r, W2r, b2r, b3d):
    return pl.pallas_call(
        _fused_body,
        grid=(NBLK,),
        in_specs=[
            pl.BlockSpec((BR, H), lambda i: (i, 0)),
            pl.BlockSpec((H, HALF), lambda i: (0, 0)),
            pl.BlockSpec((1, HALF), lambda i: (0, 0)),
            pl.BlockSpec((1, HALF), lambda i: (0, 0)),
            pl.BlockSpec((1, 1), lambda i: (0, 0)),
            pl.BlockSpec((1, 1, BR), lambda i: (i, 0, 0)),
        ],
        out_specs=[
            pl.BlockSpec((BR, 1), lambda i: (i, 0)),
            pl.BlockSpec((S, H), lambda i: (0, 0)),
        ],
        out_shape=[
            jax.ShapeDtypeStruct((NPAD, 1), jnp.float32),
            jax.ShapeDtypeStruct((S, H), jnp.float32),
        ],
    )(x, W1, b1r, W2r, b2r, b3d)


def _sc_body(e_hbm, batch_hbm, p_hbm, w_hbm, xp_hbm,
             e_v, b_v, acc_v, st_v, gd_v, p_v, shd):
    c = lax.axis_index("c")
    s = lax.axis_index("s")

    @pl.when(c == 0)
    def _work():
        base = s * CH
        pltpu.sync_copy(e_hbm.at[pl.ds(base, CH)], e_v)
        pltpu.sync_copy(batch_hbm.at[pl.ds(base, CH)], b_v)
        pltpu.sync_copy(p_hbm.at[pl.ds(s * (RPW * H), RPW * H)], p_v)

        lane = lax.iota(jnp.int32, 16)
        zero = jnp.zeros((16,), jnp.float32)
        lane_base = lane * SEGP

        def init_d(j, carry):
            off = pl.multiple_of(j * 16, 16)
            for t in range(NW):
                acc_v[pl.ds(t * SEGP + off, 16)] = zero
            return carry
        lax.fori_loop(0, SEGP // 16, init_d, 0)

        def p_acc(j, carry):
            off = pl.multiple_of(j * 16, 16)
            ev = e_v[pl.ds(off, 16)]
            bv = b_v[pl.ds(off, 16)]
            idx = lane_base + bv
            cur = plsc.load_gather(acc_v, [idx])
            plsc.store_scatter(acc_v, [idx], cur + ev)
            return carry
        lax.fori_loop(0, NV, p_acc, 0)

        def red_loc(j, carry):
            off = pl.multiple_of(j * 16, 16)
            acc = acc_v[pl.ds(off, 16)]
            for t in range(1, NW):
                acc = acc + acc_v[pl.ds(t * SEGP + off, 16)]
            gd_v[pl.ds(off, 16)] = acc
            return carry
        lax.fori_loop(0, SEGP // 16, red_loc, 0)

        pltpu.sync_copy(gd_v, shd.at[s])
        plsc.subcore_barrier()
        pltpu.sync_copy(shd, st_v)

        def red_glob(j, carry):
            off = pl.multiple_of(j * 16, 16)
            acc = st_v[0, pl.ds(off, 16)]
            for t in range(1, NW):
                acc = acc + st_v[t, pl.ds(off, 16)]
            gd_v[pl.ds(off, 16)] = acc
            return carry
        lax.fori_loop(0, SEGP // 16, red_glob, 0)

        def p_w(j, carry):
            off = pl.multiple_of(j * 16, 16)
            ev = e_v[pl.ds(off, 16)]
            bv = b_v[pl.ds(off, 16)]
            gd = plsc.load_gather(gd_v, [bv])
            wv = jnp.where(bv >= S, 0.0, ev / gd)
            e_v[pl.ds(off, 16)] = wv
            return carry
        lax.fori_loop(0, NV, p_w, 0)
        pltpu.sync_copy(e_v, w_hbm.at[pl.ds(base, CH)])

        for k in range(RPW):
            r = jnp.zeros((16,), jnp.int32) + (s * RPW + k)
            dvec = plsc.load_gather(gd_v, [r])
            scale = jnp.where(dvec > 0.0, 1.0 / dvec, 0.0)
            for t in range(H // 16):
                off = k * H + t * 16
                p_v[pl.ds(off, 16)] = p_v[pl.ds(off, 16)] * scale
        pltpu.sync_copy(p_v, xp_hbm.at[pl.ds(s * (RPW * H), RPW * H)])


@functools.cache
def _get_sc_kernel():
    return functools.partial(
        pl.kernel,
        mesh=plsc.VectorSubcoreMesh(core_axis_name="c", subcore_axis_name="s"),
        out_type=(jax.ShapeDtypeStruct((NPAD,), jnp.float32),
                  jax.ShapeDtypeStruct((S * H,), jnp.float32)),
        compiler_params=pltpu.CompilerParams(needs_layout_passes=False,
                                             use_tc_tiling_on_sc=False),
        scratch_types=[
            pltpu.VMEM((CH,), jnp.float32),
            pltpu.VMEM((CH,), jnp.int32),
            pltpu.VMEM((NW * SEGP,), jnp.float32),
            pltpu.VMEM((NW, SEGP), jnp.float32),
            pltpu.VMEM((SEGP,), jnp.float32),
            pltpu.VMEM((RPW * H,), jnp.float32),
            pltpu.VMEM_SHARED((NW, SEGP), jnp.float32),
        ],
    )(_sc_body)


def kernel(x, batch, W1, b1, W2, b2):
    b1r = b1.reshape(1, HALF)
    W2r = W2.reshape(1, HALF)
    b2r = b2.reshape(1, 1)
    batch_p = jnp.concatenate(
        [batch.astype(jnp.int32), jnp.full((NPAD - N,), S, jnp.int32)])
    e, P = _tc_fused(x, W1, b1r, W2r, b2r, batch_p.reshape(NBLK, 1, BR))
    weights, xp = _get_sc_kernel()(e.reshape(NPAD), batch_p,
                                   P.reshape(S * H))
    return xp.reshape(S, H), weights[:N]


# speedup vs baseline: 1.1484x; 1.1293x over previous
import functools

import jax
import jax.numpy as jnp
from jax import lax
from jax.experimental import pallas as pl
from jax.experimental.pallas import tpu as pltpu
from jax.experimental.pallas import tpu_sc as plsc

N = 50000
H = 256
HALF = 128
S = 512

BR = 4096
NBLK = 13
NPAD = NBLK * BR

NW = 16
CH = NPAD // NW
NV = CH // 16
SEGP = 528
RPW = S // NW


NSUB = 2
SR = BR // NSUB


def _fused_body(x_ref, w1_ref, b1_ref, w2_ref, b2_ref, b_ref, e_ref, p_ref):
    i = pl.program_id(0)

    @pl.when(i == 0)
    def _():
        p_ref[...] = jnp.zeros_like(p_ref)

    acc = jnp.zeros((S, H), jnp.float32)
    for k in range(NSUB):
        xb = x_ref[pl.ds(k * SR, SR), :]
        h = jnp.tanh(jnp.dot(xb, w1_ref[...],
                             preferred_element_type=jnp.float32) + b1_ref[...])
        sc = lax.dot_general(w2_ref[...], h, (((1,), (1,)), ((), ())),
                             preferred_element_type=jnp.float32) + b2_ref[...]
        e = jnp.exp(sc)
        e_ref[0, :, pl.ds(k * SR, SR)] = e
        b = b_ref[0, :, pl.ds(k * SR, SR)]
        seg = jax.lax.broadcasted_iota(jnp.int32, (S, SR), 0)
        coef = jnp.where(b == seg, e, 0.0).astype(jnp.bfloat16)
        acc = acc + jnp.dot(coef, xb.astype(jnp.bfloat16),
                            preferred_element_type=jnp.float32)
    p_ref[...] += acc


def _tc_fused(x, W1, b1r, W2r, b2r, b3d):
    return pl.pallas_call(
        _fused_body,
        grid=(NBLK,),
        in_specs=[
            pl.BlockSpec((BR, H), lambda i: (i, 0)),
            pl.BlockSpec((H, HALF), lambda i: (0, 0)),
            pl.BlockSpec((1, HALF), lambda i: (0, 0)),
            pl.BlockSpec((1, HALF), lambda i: (0, 0)),
            pl.BlockSpec((1, 1), lambda i: (0, 0)),
            pl.BlockSpec((1, 1, BR), lambda i: (i, 0, 0)),
        ],
        out_specs=[
            pl.BlockSpec((1, 1, BR), lambda i: (i, 0, 0)),
            pl.BlockSpec((S, H), lambda i: (0, 0)),
        ],
        out_shape=[
            jax.ShapeDtypeStruct((NBLK, 1, BR), jnp.float32),
            jax.ShapeDtypeStruct((S, H), jnp.float32),
        ],
    )(x, W1, b1r, W2r, b2r, b3d)


def _sc_body(e_hbm, batch_hbm, p_hbm, w_hbm, xp_hbm,
             e_v, b_v, acc_v, st_v, gd_v, p_v, shd):
    c = lax.axis_index("c")
    s = lax.axis_index("s")

    @pl.when(c == 0)
    def _work():
        base = s * CH
        pltpu.sync_copy(e_hbm.at[pl.ds(base, CH)], e_v)
        pltpu.sync_copy(batch_hbm.at[pl.ds(base, CH)], b_v)
        pltpu.sync_copy(p_hbm.at[pl.ds(s * (RPW * H), RPW * H)], p_v)

        lane = lax.iota(jnp.int32, 16)
        zero = jnp.zeros((16,), jnp.float32)
        lane_base = lane * SEGP

        def init_d(j, carry):
            off = pl.multiple_of(j * 16, 16)
            for t in range(NW):
                acc_v[pl.ds(t * SEGP + off, 16)] = zero
            return carry
        lax.fori_loop(0, SEGP // 16, init_d, 0)

        def p_acc(j, carry):
            for u in range(2):
                off = pl.multiple_of(j * 32 + u * 16, 16)
                ev = e_v[pl.ds(off, 16)]
                bv = b_v[pl.ds(off, 16)]
                idx = lane_base + bv
                cur = plsc.load_gather(acc_v, [idx])
                plsc.store_scatter(acc_v, [idx], cur + ev)
            return carry
        lax.fori_loop(0, NV // 2, p_acc, 0)

        def red_loc(j, carry):
            off = pl.multiple_of(j * 16, 16)
            acc = acc_v[pl.ds(off, 16)]
            for t in range(1, NW):
                acc = acc + acc_v[pl.ds(t * SEGP + off, 16)]
            gd_v[pl.ds(off, 16)] = acc
            return carry
        lax.fori_loop(0, SEGP // 16, red_loc, 0)

        pltpu.sync_copy(gd_v, shd.at[s])
        plsc.subcore_barrier()
        pltpu.sync_copy(shd, st_v)

        def red_glob(j, carry):
            off = pl.multiple_of(j * 16, 16)
            acc = st_v[0, pl.ds(off, 16)]
            for t in range(1, NW):
                acc = acc + st_v[t, pl.ds(off, 16)]
            gd_v[pl.ds(off, 16)] = acc
            return carry
        lax.fori_loop(0, SEGP // 16, red_glob, 0)

        def p_w(j, carry):
            for u in range(2):
                off = pl.multiple_of(j * 32 + u * 16, 16)
                ev = e_v[pl.ds(off, 16)]
                bv = b_v[pl.ds(off, 16)]
                gd = plsc.load_gather(gd_v, [bv])
                wv = jnp.where(bv >= S, 0.0, ev / gd)
                e_v[pl.ds(off, 16)] = wv
            return carry
        lax.fori_loop(0, NV // 2, p_w, 0)
        pltpu.sync_copy(e_v, w_hbm.at[pl.ds(base, CH)])

        for k in range(RPW):
            r = jnp.zeros((16,), jnp.int32) + (s * RPW + k)
            dvec = plsc.load_gather(gd_v, [r])
            scale = jnp.where(dvec > 0.0, 1.0 / dvec, 0.0)
            for t in range(H // 16):
                off = k * H + t * 16
                p_v[pl.ds(off, 16)] = p_v[pl.ds(off, 16)] * scale
        pltpu.sync_copy(p_v, xp_hbm.at[pl.ds(s * (RPW * H), RPW * H)])


@functools.cache
def _get_sc_kernel():
    return functools.partial(
        pl.kernel,
        mesh=plsc.VectorSubcoreMesh(core_axis_name="c", subcore_axis_name="s"),
        out_type=(jax.ShapeDtypeStruct((NPAD,), jnp.float32),
                  jax.ShapeDtypeStruct((S * H,), jnp.float32)),
        compiler_params=pltpu.CompilerParams(needs_layout_passes=False,
                                             use_tc_tiling_on_sc=False),
        scratch_types=[
            pltpu.VMEM((CH,), jnp.float32),
            pltpu.VMEM((CH,), jnp.int32),
            pltpu.VMEM((NW * SEGP,), jnp.float32),
            pltpu.VMEM((NW, SEGP), jnp.float32),
            pltpu.VMEM((SEGP,), jnp.float32),
            pltpu.VMEM((RPW * H,), jnp.float32),
            pltpu.VMEM_SHARED((NW, SEGP), jnp.float32),
        ],
    )(_sc_body)


def kernel(x, batch, W1, b1, W2, b2):
    b1r = b1.reshape(1, HALF)
    W2r = W2.reshape(1, HALF)
    b2r = b2.reshape(1, 1)
    batch_p = jnp.concatenate(
        [batch.astype(jnp.int32), jnp.full((NPAD - N,), S, jnp.int32)])
    e, P = _tc_fused(x, W1, b1r, W2r, b2r, batch_p.reshape(NBLK, 1, BR))
    weights, xp = _get_sc_kernel()(e.reshape(NPAD), batch_p,
                                   P.reshape(S * H))
    return xp.reshape(S, H), weights[:N]


# speedup vs baseline: 1.1796x; 1.0272x over previous
import functools

import jax
import jax.numpy as jnp
from jax import lax
from jax.experimental import pallas as pl
from jax.experimental.pallas import tpu as pltpu
from jax.experimental.pallas import tpu_sc as plsc

N = 50000
H = 256
HALF = 128
S = 512

BR = 4096
NBLK = 13
NPAD = NBLK * BR

NW = 16
CH = NPAD // NW
NV = CH // 16
SEGP = 528
RPW = S // NW


NSUB = 2
SR = BR // NSUB


def _fused_body(x_ref, w1_ref, b1_ref, w2_ref, b2_ref, b_ref, e_ref, p_ref):
    i = pl.program_id(0)

    @pl.when(i == 0)
    def _():
        p_ref[...] = jnp.zeros_like(p_ref)

    acc = jnp.zeros((S, H), jnp.float32)
    for k in range(NSUB):
        xb = x_ref[pl.ds(k * SR, SR), :]
        h = jnp.tanh(jnp.dot(xb, w1_ref[...],
                             preferred_element_type=jnp.float32) + b1_ref[...])
        sc = lax.dot_general(w2_ref[...], h, (((1,), (1,)), ((), ())),
                             preferred_element_type=jnp.float32) + b2_ref[...]
        e = jnp.exp(sc)
        e_ref[0, :, pl.ds(k * SR, SR)] = e
        b = b_ref[0, :, pl.ds(k * SR, SR)]
        seg = jax.lax.broadcasted_iota(jnp.int32, (S, SR), 0)
        coef = jnp.where(b == seg, e, 0.0).astype(jnp.bfloat16)
        acc = acc + jnp.dot(coef, xb.astype(jnp.bfloat16),
                            preferred_element_type=jnp.float32)
    p_ref[...] += acc


def _tc_fused(x, W1, b1r, W2r, b2r, b3d):
    return pl.pallas_call(
        _fused_body,
        grid=(NBLK,),
        in_specs=[
            pl.BlockSpec((BR, H), lambda i: (i, 0)),
            pl.BlockSpec((H, HALF), lambda i: (0, 0)),
            pl.BlockSpec((1, HALF), lambda i: (0, 0)),
            pl.BlockSpec((1, HALF), lambda i: (0, 0)),
            pl.BlockSpec((1, 1), lambda i: (0, 0)),
            pl.BlockSpec((1, 1, BR), lambda i: (i, 0, 0)),
        ],
        out_specs=[
            pl.BlockSpec((1, 1, BR), lambda i: (i, 0, 0)),
            pl.BlockSpec((S, H), lambda i: (0, 0)),
        ],
        out_shape=[
            jax.ShapeDtypeStruct((NBLK, 1, BR), jnp.float32),
            jax.ShapeDtypeStruct((S, H), jnp.float32),
        ],
    )(x, W1, b1r, W2r, b2r, b3d)


NREM = N - (NW - 1) * CH


def _sc_body(e_hbm, batch_hbm, p_hbm, w_hbm, xp_hbm,
             e_v, b_v, acc_v, acc2_v, st_v, gd_v, p_v, shd):
    c = lax.axis_index("c")
    s = lax.axis_index("s")

    @pl.when(c == 0)
    def _work():
        base = s * CH
        pltpu.sync_copy(e_hbm.at[pl.ds(base, CH)], e_v)
        pltpu.sync_copy(batch_hbm.at[pl.ds(base, CH)], b_v)
        pltpu.sync_copy(p_hbm.at[pl.ds(s * (RPW * H), RPW * H)], p_v)

        lane = lax.iota(jnp.int32, 16)
        zero = jnp.zeros((16,), jnp.float32)
        lane_base = lane * SEGP

        jlo = jnp.min(b_v[pl.ds(0, 16)]) // 16
        jhi = jnp.max(b_v[pl.ds(CH - 16, 16)]) // 16 + 1

        def init_g(j, carry):
            gd_v[pl.ds(pl.multiple_of(j * 16, 16), 16)] = zero
            return carry
        lax.fori_loop(0, SEGP // 16, init_g, 0)

        def init_d(j, carry):
            off = j * 16
            for t in range(NW):
                acc_v[pl.ds(t * SEGP + off, 16)] = zero
                acc2_v[pl.ds(t * SEGP + off, 16)] = zero
            return carry
        lax.fori_loop(jlo, jhi, init_d, 0)

        def p_acc(j, carry):
            for u, arr in ((0, acc_v), (1, acc2_v)):
                off = pl.multiple_of(j * 32 + u * 16, 16)
                ev = e_v[pl.ds(off, 16)]
                bv = b_v[pl.ds(off, 16)]
                idx = lane_base + bv
                cur = plsc.load_gather(arr, [idx])
                plsc.store_scatter(arr, [idx], cur + ev)
            return carry
        lax.fori_loop(0, NV // 2, p_acc, 0)

        def red_loc(j, carry):
            off = j * 16
            acc = acc_v[pl.ds(off, 16)] + acc2_v[pl.ds(off, 16)]
            for t in range(1, NW):
                acc = acc + (acc_v[pl.ds(t * SEGP + off, 16)] +
                             acc2_v[pl.ds(t * SEGP + off, 16)])
            gd_v[pl.ds(off, 16)] = acc
            return carry
        lax.fori_loop(jlo, jhi, red_loc, 0)

        pltpu.sync_copy(gd_v, shd.at[s])
        plsc.subcore_barrier()
        pltpu.sync_copy(shd, st_v)

        def red_glob(j, carry):
            off = j * 16
            acc = st_v[0, pl.ds(off, 16)]
            for t in range(1, NW):
                acc = acc + st_v[t, pl.ds(off, 16)]
            gd_v[pl.ds(off, 16)] = acc
            return carry
        lax.fori_loop(jlo, jhi, red_glob, 0)
        lax.fori_loop(s * (RPW // 16), (s + 1) * (RPW // 16), red_glob, 0)

        def p_w(j, carry):
            for u in range(2):
                off = pl.multiple_of(j * 32 + u * 16, 16)
                ev = e_v[pl.ds(off, 16)]
                bv = b_v[pl.ds(off, 16)]
                gd = plsc.load_gather(gd_v, [bv])
                wv = jnp.where(bv >= S, 0.0, ev / gd)
                e_v[pl.ds(off, 16)] = wv
            return carry
        lax.fori_loop(0, NV // 2, p_w, 0)

        @pl.when(s < NW - 1)
        def _wfull():
            pltpu.sync_copy(e_v, w_hbm.at[pl.ds(base, CH)])

        @pl.when(s == NW - 1)
        def _wtail():
            pltpu.sync_copy(e_v.at[pl.ds(0, NREM)],
                            w_hbm.at[pl.ds(base, NREM)])

        for k in range(RPW):
            r = jnp.zeros((16,), jnp.int32) + (s * RPW + k)
            dvec = plsc.load_gather(gd_v, [r])
            scale = jnp.where(dvec > 0.0, 1.0 / dvec, 0.0)
            for t in range(H // 16):
                off = k * H + t * 16
                p_v[pl.ds(off, 16)] = p_v[pl.ds(off, 16)] * scale
        pltpu.sync_copy(p_v, xp_hbm.at[pl.ds(s * (RPW * H), RPW * H)])


@functools.cache
def _get_sc_kernel():
    return functools.partial(
        pl.kernel,
        mesh=plsc.VectorSubcoreMesh(core_axis_name="c", subcore_axis_name="s"),
        out_type=(jax.ShapeDtypeStruct((N,), jnp.float32),
                  jax.ShapeDtypeStruct((S * H,), jnp.float32)),
        compiler_params=pltpu.CompilerParams(needs_layout_passes=False,
                                             use_tc_tiling_on_sc=False),
        scratch_types=[
            pltpu.VMEM((CH,), jnp.float32),
            pltpu.VMEM((CH,), jnp.int32),
            pltpu.VMEM((NW * SEGP,), jnp.float32),
            pltpu.VMEM((NW * SEGP,), jnp.float32),
            pltpu.VMEM((NW, SEGP), jnp.float32),
            pltpu.VMEM((SEGP,), jnp.float32),
            pltpu.VMEM((RPW * H,), jnp.float32),
            pltpu.VMEM_SHARED((NW, SEGP), jnp.float32),
        ],
    )(_sc_body)


def kernel(x, batch, W1, b1, W2, b2):
    b1r = b1.reshape(1, HALF)
    W2r = W2.reshape(1, HALF)
    b2r = b2.reshape(1, 1)
    batch_p = jnp.concatenate(
        [batch.astype(jnp.int32), jnp.full((NPAD - N,), S, jnp.int32)])
    e, P = _tc_fused(x, W1, b1r, W2r, b2r, batch_p.reshape(NBLK, 1, BR))
    weights, xp = _get_sc_kernel()(e.reshape(NPAD), batch_p,
                                   P.reshape(S * H))
    return xp.reshape(S, H), weights


# speedup vs baseline: 1.2107x; 1.0264x over previous
import functools

import jax
import jax.numpy as jnp
from jax import lax
from jax.experimental import pallas as pl
from jax.experimental.pallas import tpu as pltpu
from jax.experimental.pallas import tpu_sc as plsc

N = 50000
H = 256
HALF = 128
S = 512

BR = 4096
NBLK = 13
NPAD = NBLK * BR

NW = 16
CH = NPAD // NW
NV = CH // 16
SEGP = 528
RPW = S // NW


NSUB = 2
SR = BR // NSUB


def _fused_body(x_ref, w1_ref, b1_ref, w2_ref, b2_ref, b_ref, e_ref, p_ref):
    i = pl.program_id(0)

    @pl.when(i == 0)
    def _():
        p_ref[...] = jnp.zeros_like(p_ref)

    acc = jnp.zeros((S, H), jnp.float32)
    for k in range(NSUB):
        xb = x_ref[pl.ds(k * SR, SR), :]
        h = jnp.tanh(jnp.dot(xb, w1_ref[...],
                             preferred_element_type=jnp.float32) + b1_ref[...])
        sc = lax.dot_general(w2_ref[...], h, (((1,), (1,)), ((), ())),
                             preferred_element_type=jnp.float32) + b2_ref[...]
        e = jnp.exp(sc)
        e_ref[0, :, pl.ds(k * SR, SR)] = e
        b = b_ref[0, :, pl.ds(k * SR, SR)]
        seg = jax.lax.broadcasted_iota(jnp.int32, (S, SR), 0)
        coef = jnp.where(b == seg, e, 0.0).astype(jnp.bfloat16)
        acc = acc + jnp.dot(coef, xb.astype(jnp.bfloat16),
                            preferred_element_type=jnp.float32)
    p_ref[...] += acc


def _tc_fused(x, W1, b1r, W2r, b2r, b3d):
    return pl.pallas_call(
        _fused_body,
        grid=(NBLK,),
        in_specs=[
            pl.BlockSpec((BR, H), lambda i: (i, 0)),
            pl.BlockSpec((H, HALF), lambda i: (0, 0)),
            pl.BlockSpec((1, HALF), lambda i: (0, 0)),
            pl.BlockSpec((1, HALF), lambda i: (0, 0)),
            pl.BlockSpec((1, 1), lambda i: (0, 0)),
            pl.BlockSpec((1, 1, BR), lambda i: (i, 0, 0)),
        ],
        out_specs=[
            pl.BlockSpec((1, 1, BR), lambda i: (i, 0, 0)),
            pl.BlockSpec((S, H), lambda i: (0, 0)),
        ],
        out_shape=[
            jax.ShapeDtypeStruct((NBLK, 1, BR), jnp.float32),
            jax.ShapeDtypeStruct((S, H), jnp.float32),
        ],
    )(x, W1, b1r, W2r, b2r, b3d)


NREM = N - (NW - 1) * CH


def _sc_body(e_hbm, batch_hbm, p_hbm, w_hbm, xp_hbm,
             e_v, b_v, acc_v, acc2_v, st_v, gd_v, p_v, shd):
    c = lax.axis_index("c")
    s = lax.axis_index("s")

    @pl.when(c == 0)
    def _work():
        base = s * CH
        pltpu.sync_copy(e_hbm.at[pl.ds(base, CH)], e_v)
        pltpu.sync_copy(batch_hbm.at[pl.ds(base, CH)], b_v)
        pltpu.sync_copy(p_hbm.at[pl.ds(s * (RPW * H), RPW * H)], p_v)

        lane = lax.iota(jnp.int32, 16)
        zero = jnp.zeros((16,), jnp.float32)
        lane_base = lane * SEGP

        jlo = jnp.min(b_v[pl.ds(0, 16)]) // 16
        jhi = jnp.max(b_v[pl.ds(CH - 16, 16)]) // 16 + 1

        def init_g(j, carry):
            gd_v[pl.ds(pl.multiple_of(j * 16, 16), 16)] = zero
            return carry
        lax.fori_loop(0, SEGP // 16, init_g, 0)

        def init_d(j, carry):
            off = j * 16
            for t in range(NW):
                acc_v[pl.ds(t * SEGP + off, 16)] = zero
                acc2_v[pl.ds(t * SEGP + off, 16)] = zero
            return carry
        lax.fori_loop(jlo, jhi, init_d, 0)

        def p_acc(j, carry):
            for u, arr in ((0, acc_v), (1, acc2_v)):
                off = pl.multiple_of(j * 32 + u * 16, 16)
                ev = e_v[pl.ds(off, 16)]
                bv = b_v[pl.ds(off, 16)]
                idx = lane_base + bv
                cur = plsc.load_gather(arr, [idx])
                plsc.store_scatter(arr, [idx], cur + ev)
            return carry
        lax.fori_loop(0, NV // 2, p_acc, 0)

        def red_loc(j, carry):
            off = j * 16
            acc = acc_v[pl.ds(off, 16)] + acc2_v[pl.ds(off, 16)]
            for t in range(1, NW):
                acc = acc + (acc_v[pl.ds(t * SEGP + off, 16)] +
                             acc2_v[pl.ds(t * SEGP + off, 16)])
            gd_v[pl.ds(off, 16)] = acc
            return carry
        lax.fori_loop(jlo, jhi, red_loc, 0)

        pltpu.sync_copy(gd_v, shd.at[s])
        plsc.subcore_barrier()
        pltpu.sync_copy(shd, st_v)

        def red_glob(j, carry):
            off = j * 16
            acc = st_v[0, pl.ds(off, 16)]
            for t in range(1, NW):
                acc = acc + st_v[t, pl.ds(off, 16)]
            gd_v[pl.ds(off, 16)] = acc
            return carry
        lax.fori_loop(jlo, jhi, red_glob, 0)
        lax.fori_loop(s * (RPW // 16), (s + 1) * (RPW // 16), red_glob, 0)

        def p_w(j, carry):
            for u in range(2):
                off = pl.multiple_of(j * 32 + u * 16, 16)
                ev = e_v[pl.ds(off, 16)]
                bv = b_v[pl.ds(off, 16)]
                gd = plsc.load_gather(gd_v, [bv])
                wv = jnp.where(bv >= S, 0.0, ev / gd)
                e_v[pl.ds(off, 16)] = wv
            return carry
        lax.fori_loop(0, NV // 2, p_w, 0)

        @pl.when(s < NW - 1)
        def _wfull():
            pltpu.sync_copy(e_v, w_hbm.at[pl.ds(base, CH)])

        @pl.when(s == NW - 1)
        def _wtail():
            pltpu.sync_copy(e_v.at[pl.ds(0, NREM)],
                            w_hbm.at[pl.ds(base, NREM)])

        def p_scale(k, carry):
            r = jnp.zeros((16,), jnp.int32) + (s * RPW + k)
            dvec = plsc.load_gather(gd_v, [r])
            scale = jnp.where(dvec > 0.0, 1.0 / dvec, 0.0)
            for t in range(H // 16):
                off = k * H + t * 16
                p_v[pl.ds(off, 16)] = p_v[pl.ds(off, 16)] * scale
            return carry
        lax.fori_loop(0, RPW, p_scale, 0)
        pltpu.sync_copy(p_v, xp_hbm.at[pl.ds(s * (RPW * H), RPW * H)])


@functools.cache
def _get_sc_kernel():
    return functools.partial(
        pl.kernel,
        mesh=plsc.VectorSubcoreMesh(core_axis_name="c", subcore_axis_name="s",
                                    num_cores=1),
        out_type=(jax.ShapeDtypeStruct((N,), jnp.float32),
                  jax.ShapeDtypeStruct((S * H,), jnp.float32)),
        compiler_params=pltpu.CompilerParams(needs_layout_passes=False,
                                             use_tc_tiling_on_sc=False),
        scratch_types=[
            pltpu.VMEM((CH,), jnp.float32),
            pltpu.VMEM((CH,), jnp.int32),
            pltpu.VMEM((NW * SEGP,), jnp.float32),
            pltpu.VMEM((NW * SEGP,), jnp.float32),
            pltpu.VMEM((NW, SEGP), jnp.float32),
            pltpu.VMEM((SEGP,), jnp.float32),
            pltpu.VMEM((RPW * H,), jnp.float32),
            pltpu.VMEM_SHARED((NW, SEGP), jnp.float32),
        ],
    )(_sc_body)


def kernel(x, batch, W1, b1, W2, b2):
    b1r = b1.reshape(1, HALF)
    W2r = W2.reshape(1, HALF)
    b2r = b2.reshape(1, 1)
    batch_p = jnp.concatenate(
        [batch.astype(jnp.int32), jnp.full((NPAD - N,), S, jnp.int32)])
    e, P = _tc_fused(x, W1, b1r, W2r, b2r, batch_p.reshape(NBLK, 1, BR))
    weights, xp = _get_sc_kernel()(e.reshape(NPAD), batch_p,
                                   P.reshape(S * H))
    return xp.reshape(S, H), weights
